# baseline (speedup 1.0000x reference)
import functools

import jax
import jax.numpy as jnp
from jax import lax
from jax.experimental import pallas as pl
from jax.experimental.pallas import tpu as pltpu
from jax.experimental.pallas import tpu_sc as plsc

F = 128
H = 64
W = 80
NTILES = 16
NCORES = 2
NW = NTILES * NCORES
CK = 112


def _cdiv(a, b):
  return (a + b - 1) // b


def _hsum(v, perms):
  for p in perms:
    v = v + v.at[p].get(mode="promise_in_bounds")
  return v


def _zero_ref(ref, rows, width, zeros16):
  for j in range(width // 16):
    ref[rows, pl.ds(16 * j, 16)] = zeros16


def _make_edge_pass1(n_rows, e_pad, n_tab):
  per_w = e_pad // NTILES
  n_chunks = per_w // CK
  rows_per_tile = n_rows // NTILES
  n_zchunks = rows_per_tile // 64

  mesh = plsc.VectorSubcoreMesh(core_axis_name="c", subcore_axis_name="s")

  @functools.partial(
      pl.kernel,
      out_type=jax.ShapeDtypeStruct((NCORES, n_rows, W), jnp.float32),
      mesh=mesh,
      compiler_params=pltpu.CompilerParams(use_tc_tiling_on_sc=False),
      scratch_types=[
          pltpu.VMEM((CK,), jnp.int32),
          pltpu.VMEM((CK,), jnp.int32),
          pltpu.VMEM((CK,), jnp.int32),
          pltpu.VMEM((CK,), jnp.int32),
          pltpu.VMEM((CK,), jnp.int32),
          pltpu.VMEM((CK,), jnp.int32),
          pltpu.VMEM((CK, H), jnp.float32),
          pltpu.VMEM((CK, H), jnp.float32),
          pltpu.VMEM((CK, H), jnp.float32),
          pltpu.VMEM((CK, H), jnp.float32),
          pltpu.VMEM((CK, W), jnp.float32),
          pltpu.VMEM((64, W), jnp.float32),
          pltpu.VMEM((NCORES, H), jnp.float32),
          pltpu.VMEM_SHARED((n_rows, W), jnp.float32),
          pltpu.SemaphoreType.DMA,
          pltpu.SemaphoreType.DMA,
      ],
  )
  def edge_pass(xlh_hbm, xrh_hbm, att_hbm, src_hbm, dst_hbm, acc_out,
                srcvA, dstvA, dstgA, srcvB, dstvB, dstgB,
                xlvA, xrvA, xlvB, xrvB, orow, zbuf, attv, acc_sh, semA, semB):
    c = lax.axis_index("c")
    s = lax.axis_index("s")
    zeros16 = jnp.zeros((16,), jnp.float32)
    iot = lax.iota(jnp.int32, 16)
    perms = [jnp.bitwise_xor(iot, jnp.full((16,), 1 << k, jnp.int32))
             for k in range(4)]

    def zrow(r, carry):
      _zero_ref(zbuf, r, W, zeros16)
      return carry
    lax.fori_loop(0, 64, zrow, 0)

    tb = s * rows_per_tile
    def zchunk(k, carry):
      pltpu.sync_copy(zbuf, acc_sh.at[pl.ds(tb + 64 * k, 64)])
      return carry
    lax.fori_loop(0, n_zchunks, zchunk, 0)

    pltpu.sync_copy(att_hbm, attv)
    plsc.subcore_barrier()

    base = s * per_w
    coff = jnp.full((16,), 1, jnp.int32) * (c * n_tab)
    att_r = [attv[c, pl.ds(16 * j, 16)] for j in range(4)]
    last = n_chunks - 1

    def fetch(i, srcv, dstv, dstgv, xlv, xrv, sem):
      off = base + i * CK
      pltpu.sync_copy(src_hbm.at[pl.ds(off, CK)], srcv)
      pltpu.sync_copy(dst_hbm.at[pl.ds(off, CK)], dstv)
      for g in range(CK // 16):
        srcv[pl.ds(16 * g, 16)] = srcv[pl.ds(16 * g, 16)] + coff
        dstgv[pl.ds(16 * g, 16)] = dstv[pl.ds(16 * g, 16)] + coff
      pltpu.async_copy(xlh_hbm.at[srcv], xlv, sem)
      pltpu.async_copy(xrh_hbm.at[dstgv], xrv, sem)

    def waitg(srcv, dstgv, xlv, xrv, sem):
      pltpu.make_async_copy(xlh_hbm.at[srcv], xlv, sem).wait()
      pltpu.make_async_copy(xrh_hbm.at[dstgv], xrv, sem).wait()

    def compute(dstv, xlv, xrv):
      def edge(e, carry2):
        xs = [xlv[e, pl.ds(16 * j, 16)] for j in range(4)]
        s0 = zeros16
        for j in range(4):
          t = xs[j] + xrv[e, pl.ds(16 * j, 16)]
          s0 = s0 + jnp.where(t >= 0, t, 0.2 * t) * att_r[j]
        p0 = jnp.exp(_hsum(s0, perms))
        for j in range(4):
          orow[e, pl.ds(16 * j, 16)] = p0 * xs[j]
        orow[e, pl.ds(H, 16)] = jnp.where(iot == 0, p0, zeros16)
        return carry2

      lax.fori_loop(0, CK, edge, 0)
      pltpu.sync_copy(orow, acc_sh.at[dstv], add=True)

    fetch(0, srcvA, dstvA, dstgA, xlvA, xrvA, semA)

    def chunk2(kk, carry):
      k = 2 * kk
      fetch(k + 1, srcvB, dstvB, dstgB, xlvB, xrvB, semB)
      waitg(srcvA, dstgA, xlvA, xrvA, semA)
      compute(dstvA, xlvA, xrvA)
      fetch(jnp.minimum(k + 2, last), srcvA, dstvA, dstgA, xlvA, xrvA, semA)
      waitg(srcvB, dstgB, xlvB, xrvB, semB)
      compute(dstvB, xlvB, xrvB)
      return carry

    lax.fori_loop(0, n_chunks // 2, chunk2, 0)
    waitg(srcvA, dstgA, xlvA, xrvA, semA)
    plsc.subcore_barrier()

    pltpu.sync_copy(acc_sh.at[pl.ds(tb, rows_per_tile)],
                    acc_out.at[c, pl.ds(tb, rows_per_tile)])

  return edge_pass


def _make_edge_pass2(n_rows, e_pad):
  per_w = e_pad // NW
  n_chunks = per_w // CK
  rows_per_tile = n_rows // NTILES
  n_zchunks = rows_per_tile // 64

  mesh = plsc.VectorSubcoreMesh(core_axis_name="c", subcore_axis_name="s")

  @functools.partial(
      pl.kernel,
      out_type=[
          jax.ShapeDtypeStruct((NCORES, n_rows, W), jnp.float32),
          jax.ShapeDtypeStruct((NCORES, n_rows, W), jnp.float32),
          jax.ShapeDtypeStruct((e_pad,), jnp.float32),
      ],
      mesh=mesh,
      compiler_params=pltpu.CompilerParams(use_tc_tiling_on_sc=False),
      scratch_types=[
          pltpu.VMEM((CK,), jnp.int32),
          pltpu.VMEM((CK,), jnp.int32),
          pltpu.VMEM((CK,), jnp.int32),
          pltpu.VMEM((CK,), jnp.int32),
          pltpu.VMEM((CK, H), jnp.float32),
          pltpu.VMEM((CK, H), jnp.float32),
          pltpu.VMEM((CK, F), jnp.float32),
          pltpu.VMEM((CK, H), jnp.float32),
          pltpu.VMEM((CK, H), jnp.float32),
          pltpu.VMEM((CK, F), jnp.float32),
          pltpu.VMEM((CK, W), jnp.float32),
          pltpu.VMEM((CK,), jnp.float32),
          pltpu.VMEM((CK,), jnp.float32),
          pltpu.VMEM((CK,), jnp.float32),
          pltpu.VMEM((64, W), jnp.float32),
          pltpu.VMEM((1, F), jnp.float32),
          pltpu.VMEM_SHARED((n_rows, W), jnp.float32),
          pltpu.SemaphoreType.DMA,
          pltpu.SemaphoreType.DMA,
      ],
  )
  def edge_pass(xlo_hbm, xhi_hbm, xr_hbm, att_hbm, src_hbm, dst_hbm,
                acc_lo_out, acc_hi_out, p_out,
                srcvA, dstvA, srcvB, dstvB, xloA, xhiA, xrvA, xloB, xhiB,
                xrvB, orow, pvals, pvA, pvB, zbuf, attv, acc_sh, semA, semB):
    c = lax.axis_index("c")
    s = lax.axis_index("s")
    wid = c * NTILES + s
    zeros16 = jnp.zeros((16,), jnp.float32)
    iot = lax.iota(jnp.int32, 16)
    perms = [jnp.bitwise_xor(iot, jnp.full((16,), 1 << k, jnp.int32))
             for k in range(4)]

    def zrow(r, carry):
      _zero_ref(zbuf, r, W, zeros16)
      return carry
    lax.fori_loop(0, 64, zrow, 0)

    tb = s * rows_per_tile
    def zchunk(k, carry):
      pltpu.sync_copy(zbuf, acc_sh.at[pl.ds(tb + 64 * k, 64)])
      return carry
    lax.fori_loop(0, n_zchunks, zchunk, 0)

    pltpu.sync_copy(att_hbm, attv)
    plsc.subcore_barrier()

    base = wid * per_w
    att_r = [attv[0, pl.ds(16 * j, 16)] for j in range(8)]
    last = n_chunks - 1

    def fetch_a(i, srcv, dstv, xlov, xhiv, xrv, sem):
      off = base + i * CK
      pltpu.sync_copy(src_hbm.at[pl.ds(off, CK)], srcv)
      pltpu.sync_copy(dst_hbm.at[pl.ds(off, CK)], dstv)
      pltpu.async_copy(xlo_hbm.at[srcv], xlov, sem)
      pltpu.async_copy(xhi_hbm.at[srcv], xhiv, sem)
      pltpu.async_copy(xr_hbm.at[dstv], xrv, sem)

    def wait_a(srcv, dstv, xlov, xhiv, xrv, sem):
      pltpu.make_async_copy(xlo_hbm.at[srcv], xlov, sem).wait()
      pltpu.make_async_copy(xhi_hbm.at[srcv], xhiv, sem).wait()
      pltpu.make_async_copy(xr_hbm.at[dstv], xrv, sem).wait()

    def compute_a(i, dstv, xlov, xhiv, xrv):
      off = base + i * CK

      def group(g, carry2):
        pacc = zeros16
        for u in range(16):
          e = 16 * g + u
          xs = [xlov[e, pl.ds(16 * j, 16)] for j in range(4)]
          s0 = zeros16
          for j in range(4):
            t = xs[j] + xrv[e, pl.ds(16 * j, 16)]
            s0 = s0 + jnp.where(t >= 0, t, 0.2 * t) * att_r[j]
          for j in range(4):
            t = xhiv[e, pl.ds(16 * j, 16)] + xrv[e, pl.ds(64 + 16 * j, 16)]
            s0 = s0 + jnp.where(t >= 0, t, 0.2 * t) * att_r[4 + j]
          p0 = jnp.exp(_hsum(s0, perms))
          for j in range(4):
            orow[e, pl.ds(16 * j, 16)] = p0 * xs[j]
          orow[e, pl.ds(H, 16)] = jnp.where(iot == 0, p0, zeros16)
          pacc = jnp.where(iot == u, p0, pacc)
        pvals[pl.ds(16 * g, 16)] = pacc
        return carry2

      lax.fori_loop(0, CK // 16, group, 0)
      pltpu.sync_copy(orow, acc_sh.at[dstv], add=True)
      pltpu.sync_copy(pvals, p_out.at[pl.ds(off, CK)])

    fetch_a(0, srcvA, dstvA, xloA, xhiA, xrvA, semA)

    def chunk2_a(kk, carry):
      k = 2 * kk
      fetch_a(k + 1, srcvB, dstvB, xloB, xhiB, xrvB, semB)
      wait_a(srcvA, dstvA, xloA, xhiA, xrvA, semA)
      compute_a(k, dstvA, xloA, xhiA, xrvA)
      fetch_a(jnp.minimum(k + 2, last), srcvA, dstvA, xloA, xhiA, xrvA, semA)
      wait_a(srcvB, dstvB, xloB, xhiB, xrvB, semB)
      compute_a(k + 1, dstvB, xloB, xhiB, xrvB)
      return carry

    lax.fori_loop(0, n_chunks // 2, chunk2_a, 0)
    wait_a(srcvA, dstvA, xloA, xhiA, xrvA, semA)
    plsc.subcore_barrier()

    pltpu.sync_copy(acc_sh.at[pl.ds(tb, rows_per_tile)],
                    acc_lo_out.at[c, pl.ds(tb, rows_per_tile)])

    def zacc(k, carry):
      pltpu.sync_copy(zbuf, acc_sh.at[pl.ds(tb + 64 * k, 64)])
      return carry
    lax.fori_loop(0, n_zchunks, zacc, 0)
    plsc.subcore_barrier()

    lane_sel = [jnp.full((16,), u, jnp.int32) for u in range(16)]

    def fetch_b(i, srcv, dstv, xhiv, pv, sem):
      off = base + i * CK
      pltpu.sync_copy(src_hbm.at[pl.ds(off, CK)], srcv)
      pltpu.sync_copy(dst_hbm.at[pl.ds(off, CK)], dstv)
      pltpu.async_copy(xhi_hbm.at[srcv], xhiv, sem)
      pltpu.async_copy(p_out.at[pl.ds(off, CK)], pv, sem)

    def wait_b(srcv, xhiv, pv, sem):
      pltpu.make_async_copy(xhi_hbm.at[srcv], xhiv, sem).wait()
      pltpu.make_async_copy(p_out.at[pl.ds(0, CK)], pv, sem).wait()

    def compute_b(dstv, xhiv, pv):
      def group(g, carry2):
        pg = pv[pl.ds(16 * g, 16)]
        for u in range(16):
          e = 16 * g + u
          p0 = pg.at[lane_sel[u]].get(mode="promise_in_bounds")
          for j in range(4):
            orow[e, pl.ds(16 * j, 16)] = p0 * xhiv[e, pl.ds(16 * j, 16)]
        return carry2

      lax.fori_loop(0, CK // 16, group, 0)
      pltpu.sync_copy(orow, acc_sh.at[dstv], add=True)

    fetch_b(0, srcvA, dstvA, xhiA, pvA, semA)

    def chunk2_b(kk, carry):
      k = 2 * kk
      fetch_b(k + 1, srcvB, dstvB, xhiB, pvB, semB)
      wait_b(srcvA, xhiA, pvA, semA)
      compute_b(dstvA, xhiA, pvA)
      fetch_b(jnp.minimum(k + 2, last), srcvA, dstvA, xhiA, pvA, semA)
      wait_b(srcvB, xhiB, pvB, semB)
      compute_b(dstvB, xhiB, pvB)
      return carry

    lax.fori_loop(0, n_chunks // 2, chunk2_b, 0)
    wait_b(srcvA, xhiA, pvA, semA)
    plsc.subcore_barrier()

    pltpu.sync_copy(acc_sh.at[pl.ds(tb, rows_per_tile)],
                    acc_hi_out.at[c, pl.ds(tb, rows_per_tile)])

  return edge_pass


def _head_matmul(x, wa, wb):
  m = x.shape[0]
  bm = 1000
  grid = m // bm

  def body(x_ref, wa_ref, wb_ref, oa_ref, ob_ref):
    xv = x_ref[...]
    wa_v = wa_ref[...]
    wb_v = wb_ref[...]
    oa_ref[0] = jnp.dot(xv, wa_v[:, :H], preferred_element_type=jnp.float32)
    oa_ref[1] = jnp.dot(xv, wa_v[:, H:], preferred_element_type=jnp.float32)
    ob_ref[0] = jnp.dot(xv, wb_v[:, :H], preferred_element_type=jnp.float32)
    ob_ref[1] = jnp.dot(xv, wb_v[:, H:], preferred_element_type=jnp.float32)

  return pl.pallas_call(
      body,
      grid=(grid,),
      in_specs=[
          pl.BlockSpec((bm, F), lambda i: (i, 0)),
          pl.BlockSpec((F, F), lambda i: (0, 0)),
          pl.BlockSpec((F, F), lambda i: (0, 0)),
      ],
      out_specs=[
          pl.BlockSpec((2, bm, H), lambda i: (0, i, 0)),
          pl.BlockSpec((2, bm, H), lambda i: (0, i, 0)),
      ],
      out_shape=[
          jax.ShapeDtypeStruct((2, m, H), jnp.float32),
          jax.ShapeDtypeStruct((2, m, H), jnp.float32),
      ],
  )(x, wa, wb)


def _combine_matmul(acc, bias, wa, wb, m):
  bm = 1000
  grid = m // bm

  def body(acc_ref, b_ref, wa_ref, wb_ref, olo_ref, ohi_ref, ob_ref):
    a = jnp.concatenate([acc_ref[0, :, :H], acc_ref[1, :, :H]], axis=1)
    d0 = jnp.broadcast_to(acc_ref[0, :, H:H + 1], (bm, H))
    d1 = jnp.broadcast_to(acc_ref[1, :, H:H + 1], (bm, H))
    d = jnp.concatenate([d0, d1], axis=1)
    h = a / (d + 1e-16) + b_ref[...]
    h = jnp.maximum(h, 0.0)
    xl = jnp.dot(h, wa_ref[...], preferred_element_type=jnp.float32)
    olo_ref[...] = xl[:, :H]
    ohi_ref[...] = xl[:, H:]
    ob_ref[...] = jnp.dot(h, wb_ref[...], preferred_element_type=jnp.float32)

  return pl.pallas_call(
      body,
      grid=(grid,),
      in_specs=[
          pl.BlockSpec((2, bm, W), lambda i: (0, i, 0)),
          pl.BlockSpec((1, F), lambda i: (0, 0)),
          pl.BlockSpec((F, F), lambda i: (0, 0)),
          pl.BlockSpec((F, F), lambda i: (0, 0)),
      ],
      out_specs=[
          pl.BlockSpec((bm, H), lambda i: (i, 0)),
          pl.BlockSpec((bm, H), lambda i: (i, 0)),
          pl.BlockSpec((bm, F), lambda i: (i, 0)),
      ],
      out_shape=[
          jax.ShapeDtypeStruct((m, H), jnp.float32),
          jax.ShapeDtypeStruct((m, H), jnp.float32),
          jax.ShapeDtypeStruct((m, F), jnp.float32),
      ],
  )(acc, bias, wa, wb)


def _finale(acc_lo, acc_hi, bias, we1, be1, we2, be2, m):
  bm = 1000
  grid = m // bm

  def body(alo_ref, ahi_ref, b_ref, w1_ref, b1_ref, w2_ref, b2_ref, o_ref):
    a = jnp.concatenate([alo_ref[0, :, :H] + alo_ref[1, :, :H],
                         ahi_ref[0, :, :H] + ahi_ref[1, :, :H]], axis=1)
    d = alo_ref[0, :, H:H + 1] + alo_ref[1, :, H:H + 1]
    h = a / (jnp.broadcast_to(d, (bm, F)) + 1e-16) + b_ref[...]
    hmax = jnp.max(h, axis=1, keepdims=True)
    ex = jnp.exp(h - hmax)
    ls = h - hmax - jnp.log(jnp.sum(ex, axis=1, keepdims=True))
    r = jnp.dot(ls, w1_ref[...], preferred_element_type=jnp.float32) + b1_ref[...]
    r = jnp.maximum(r, 0.0)
    r = jnp.dot(r, w2_ref[...], preferred_element_type=jnp.float32) + b2_ref[...]
    r = jax.nn.sigmoid(r)
    o_ref[...] = ls + r

  return pl.pallas_call(
      body,
      grid=(grid,),
      in_specs=[
          pl.BlockSpec((2, bm, W), lambda i: (0, i, 0)),
          pl.BlockSpec((2, bm, W), lambda i: (0, i, 0)),
          pl.BlockSpec((1, F), lambda i: (0, 0)),
          pl.BlockSpec((F, F), lambda i: (0, 0)),
          pl.BlockSpec((1, F), lambda i: (0, 0)),
          pl.BlockSpec((F, F), lambda i: (0, 0)),
          pl.BlockSpec((1, F), lambda i: (0, 0)),
      ],
      out_specs=pl.BlockSpec((bm, F), lambda i: (i, 0)),
      out_shape=jax.ShapeDtypeStruct((m, F), jnp.float32),
  )(acc_lo, acc_hi, bias, we1, be1, we2, be2)


def kernel(x, edge_index, Wl1, Wr1, att1, b1, Wl2, Wr2, att2, b2,
           We1, be1, We2, be2):
  n = x.shape[0]
  e = edge_index.shape[1]
  n_edges = e + n
  e_pad = _cdiv(n_edges, NW * CK * 2) * (NW * CK * 2)
  n_rows = _cdiv(n + 1, NTILES * 64) * (NTILES * 64)

  loop = jnp.arange(n, dtype=jnp.int32)
  pad = e_pad - n_edges
  src = jnp.concatenate([edge_index[0], loop,
                         jnp.zeros((pad,), jnp.int32)])
  dst = jnp.concatenate([edge_index[1], loop,
                         jnp.full((pad,), n, jnp.int32)])

  edge1 = _make_edge_pass1(n_rows, e_pad, n)
  edge2 = _make_edge_pass2(n_rows, e_pad)

  xlh, xrh = _head_matmul(x, Wl1, Wr1)
  acc1 = edge1(xlh.reshape(2 * n, H), xrh.reshape(2 * n, H), att1, src, dst)
  xl2_lo, xl2_hi, xr2 = _combine_matmul(acc1[:, :n], b1.reshape(1, F),
                                        Wl2, Wr2, n)
  acc_lo, acc_hi, _ = edge2(xl2_lo, xl2_hi, xr2, att2.reshape(1, F), src, dst)
  return _finale(acc_lo[:, :n], acc_hi[:, :n], b2.reshape(1, F),
                 We1, be1.reshape(1, F), We2, be2.reshape(1, F), n)


# speedup vs baseline: 1.4525x; 1.4525x over previous
import functools

import jax
import jax.numpy as jnp
from jax import lax
from jax.experimental import pallas as pl
from jax.experimental.pallas import tpu as pltpu
from jax.experimental.pallas import tpu_sc as plsc

F = 128
H = 64
W = 80
NTILES = 16
NCORES = 2
NW = NTILES * NCORES
CK = 96


def _cdiv(a, b):
  return (a + b - 1) // b


def _hsum(v, perms):
  for p in perms:
    v = v + v.at[p].get(mode="promise_in_bounds")
  return v


def _zero_ref(ref, rows, width, zeros16):
  for j in range(width // 16):
    ref[rows, pl.ds(16 * j, 16)] = zeros16


def _make_edge_pass1(n_rows, e_pad, n_tab):
  per_w = e_pad // NTILES
  n_chunks = per_w // CK
  rows_per_tile = n_rows // NTILES
  n_zchunks = rows_per_tile // 64

  mesh = plsc.VectorSubcoreMesh(core_axis_name="c", subcore_axis_name="s")

  @functools.partial(
      pl.kernel,
      out_type=jax.ShapeDtypeStruct((NCORES, n_rows, W), jnp.float32),
      mesh=mesh,
      compiler_params=pltpu.CompilerParams(use_tc_tiling_on_sc=False),
      scratch_types=[
          pltpu.VMEM((CK,), jnp.int32),
          pltpu.VMEM((CK,), jnp.int32),
          pltpu.VMEM((CK,), jnp.int32),
          pltpu.VMEM((CK,), jnp.int32),
          pltpu.VMEM((CK,), jnp.int32),
          pltpu.VMEM((CK,), jnp.int32),
          pltpu.VMEM((CK, H), jnp.float32),
          pltpu.VMEM((CK, H), jnp.float32),
          pltpu.VMEM((CK, H), jnp.float32),
          pltpu.VMEM((CK, H), jnp.float32),
          pltpu.VMEM((CK, W), jnp.float32),
          pltpu.VMEM((64, W), jnp.float32),
          pltpu.VMEM((NCORES, H), jnp.float32),
          pltpu.VMEM_SHARED((n_rows, W), jnp.float32),
          pltpu.SemaphoreType.DMA,
          pltpu.SemaphoreType.DMA,
      ],
  )
  def edge_pass(xlh_hbm, xrh_hbm, att_hbm, src_hbm, dst_hbm, acc_out,
                srcvA, dstvA, dstgA, srcvB, dstvB, dstgB,
                xlvA, xrvA, xlvB, xrvB, orow, zbuf, attv, acc_sh, semA, semB):
    c = lax.axis_index("c")
    s = lax.axis_index("s")
    zeros16 = jnp.zeros((16,), jnp.float32)
    iot = lax.iota(jnp.int32, 16)
    perms = [jnp.bitwise_xor(iot, jnp.full((16,), 1 << k, jnp.int32))
             for k in range(4)]

    def zrow(r, carry):
      _zero_ref(zbuf, r, W, zeros16)
      return carry
    lax.fori_loop(0, 64, zrow, 0)

    tb = s * rows_per_tile
    def zchunk(k, carry):
      pltpu.sync_copy(zbuf, acc_sh.at[pl.ds(tb + 64 * k, 64)])
      return carry
    lax.fori_loop(0, n_zchunks, zchunk, 0)

    pltpu.sync_copy(att_hbm, attv)
    plsc.subcore_barrier()

    base = s * per_w
    coff = jnp.full((16,), 1, jnp.int32) * (c * n_tab)
    att_r = [attv[c, pl.ds(16 * j, 16)] for j in range(4)]
    last = n_chunks - 1

    def fetch(i, srcv, dstv, dstgv, xlv, xrv, sem):
      off = base + i * CK
      pltpu.sync_copy(src_hbm.at[pl.ds(off, CK)], srcv)
      pltpu.sync_copy(dst_hbm.at[pl.ds(off, CK)], dstv)
      for g in range(CK // 16):
        srcv[pl.ds(16 * g, 16)] = srcv[pl.ds(16 * g, 16)] + coff
        dstgv[pl.ds(16 * g, 16)] = dstv[pl.ds(16 * g, 16)] + coff
      pltpu.async_copy(xlh_hbm.at[srcv], xlv, sem)
      pltpu.async_copy(xrh_hbm.at[dstgv], xrv, sem)

    def waitg(srcv, dstgv, xlv, xrv, sem):
      pltpu.make_async_copy(xlh_hbm.at[srcv], xlv, sem).wait()
      pltpu.make_async_copy(xrh_hbm.at[dstgv], xrv, sem).wait()

    def compute(dstv, xlv, xrv):
      @plsc.parallel_loop(0, CK, unroll=4)
      def edge(e):
        xs = [xlv[e, pl.ds(16 * j, 16)] for j in range(4)]
        s0 = zeros16
        for j in range(4):
          t = xs[j] + xrv[e, pl.ds(16 * j, 16)]
          s0 = s0 + jnp.where(t >= 0, t, 0.2 * t) * att_r[j]
        p0 = jnp.exp(_hsum(s0, perms))
        for j in range(4):
          orow[e, pl.ds(16 * j, 16)] = p0 * xs[j]
        orow[e, pl.ds(H, 16)] = jnp.where(iot == 0, p0, zeros16)

      pltpu.sync_copy(orow, acc_sh.at[dstv], add=True)

    fetch(0, srcvA, dstvA, dstgA, xlvA, xrvA, semA)

    def chunk2(kk, carry):
      k = 2 * kk
      fetch(k + 1, srcvB, dstvB, dstgB, xlvB, xrvB, semB)
      waitg(srcvA, dstgA, xlvA, xrvA, semA)
      compute(dstvA, xlvA, xrvA)
      fetch(jnp.minimum(k + 2, last), srcvA, dstvA, dstgA, xlvA, xrvA, semA)
      waitg(srcvB, dstgB, xlvB, xrvB, semB)
      compute(dstvB, xlvB, xrvB)
      return carry

    lax.fori_loop(0, n_chunks // 2, chunk2, 0)
    waitg(srcvA, dstgA, xlvA, xrvA, semA)
    plsc.subcore_barrier()

    pltpu.sync_copy(acc_sh.at[pl.ds(tb, rows_per_tile)],
                    acc_out.at[c, pl.ds(tb, rows_per_tile)])

  return edge_pass


def _make_edge_pass2(n_rows, e_pad):
  per_w = e_pad // NW
  n_chunks = per_w // CK
  rows_per_tile = n_rows // NTILES
  n_zchunks = rows_per_tile // 64

  mesh = plsc.VectorSubcoreMesh(core_axis_name="c", subcore_axis_name="s")

  @functools.partial(
      pl.kernel,
      out_type=[
          jax.ShapeDtypeStruct((NCORES, n_rows, W), jnp.float32),
          jax.ShapeDtypeStruct((NCORES, n_rows, W), jnp.float32),
          jax.ShapeDtypeStruct((e_pad,), jnp.float32),
      ],
      mesh=mesh,
      compiler_params=pltpu.CompilerParams(use_tc_tiling_on_sc=False),
      scratch_types=[
          pltpu.VMEM((CK,), jnp.int32),
          pltpu.VMEM((CK,), jnp.int32),
          pltpu.VMEM((CK,), jnp.int32),
          pltpu.VMEM((CK,), jnp.int32),
          pltpu.VMEM((CK, H), jnp.float32),
          pltpu.VMEM((CK, H), jnp.float32),
          pltpu.VMEM((CK, F), jnp.float32),
          pltpu.VMEM((CK, H), jnp.float32),
          pltpu.VMEM((CK, H), jnp.float32),
          pltpu.VMEM((CK, F), jnp.float32),
          pltpu.VMEM((CK, W), jnp.float32),
          pltpu.VMEM((CK,), jnp.float32),
          pltpu.VMEM((CK,), jnp.float32),
          pltpu.VMEM((CK,), jnp.float32),
          pltpu.VMEM((64, W), jnp.float32),
          pltpu.VMEM((1, F), jnp.float32),
          pltpu.VMEM_SHARED((n_rows, W), jnp.float32),
          pltpu.SemaphoreType.DMA,
          pltpu.SemaphoreType.DMA,
      ],
  )
  def edge_pass(xlo_hbm, xhi_hbm, xr_hbm, att_hbm, src_hbm, dst_hbm,
                acc_lo_out, acc_hi_out, p_out,
                srcvA, dstvA, srcvB, dstvB, xloA, xhiA, xrvA, xloB, xhiB,
                xrvB, orow, pvals, pvA, pvB, zbuf, attv, acc_sh, semA, semB):
    c = lax.axis_index("c")
    s = lax.axis_index("s")
    wid = c * NTILES + s
    zeros16 = jnp.zeros((16,), jnp.float32)
    iot = lax.iota(jnp.int32, 16)
    perms = [jnp.bitwise_xor(iot, jnp.full((16,), 1 << k, jnp.int32))
             for k in range(4)]

    def zrow(r, carry):
      _zero_ref(zbuf, r, W, zeros16)
      return carry
    lax.fori_loop(0, 64, zrow, 0)

    tb = s * rows_per_tile
    def zchunk(k, carry):
      pltpu.sync_copy(zbuf, acc_sh.at[pl.ds(tb + 64 * k, 64)])
      return carry
    lax.fori_loop(0, n_zchunks, zchunk, 0)

    pltpu.sync_copy(att_hbm, attv)
    plsc.subcore_barrier()

    base = wid * per_w
    att_r = [attv[0, pl.ds(16 * j, 16)] for j in range(8)]
    last = n_chunks - 1

    def fetch_a(i, srcv, dstv, xlov, xhiv, xrv, sem):
      off = base + i * CK
      pltpu.sync_copy(src_hbm.at[pl.ds(off, CK)], srcv)
      pltpu.sync_copy(dst_hbm.at[pl.ds(off, CK)], dstv)
      pltpu.async_copy(xlo_hbm.at[srcv], xlov, sem)
      pltpu.async_copy(xhi_hbm.at[srcv], xhiv, sem)
      pltpu.async_copy(xr_hbm.at[dstv], xrv, sem)

    def wait_a(srcv, dstv, xlov, xhiv, xrv, sem):
      pltpu.make_async_copy(xlo_hbm.at[srcv], xlov, sem).wait()
      pltpu.make_async_copy(xhi_hbm.at[srcv], xhiv, sem).wait()
      pltpu.make_async_copy(xr_hbm.at[dstv], xrv, sem).wait()

    def compute_a(i, dstv, xlov, xhiv, xrv):
      off = base + i * CK

      @plsc.parallel_loop(0, CK // 16)
      def group(g):
        pacc = zeros16
        for u in range(16):
          e = 16 * g + u
          xs = [xlov[e, pl.ds(16 * j, 16)] for j in range(4)]
          s0 = zeros16
          for j in range(4):
            t = xs[j] + xrv[e, pl.ds(16 * j, 16)]
            s0 = s0 + jnp.where(t >= 0, t, 0.2 * t) * att_r[j]
          for j in range(4):
            t = xhiv[e, pl.ds(16 * j, 16)] + xrv[e, pl.ds(64 + 16 * j, 16)]
            s0 = s0 + jnp.where(t >= 0, t, 0.2 * t) * att_r[4 + j]
          p0 = jnp.exp(_hsum(s0, perms))
          for j in range(4):
            orow[e, pl.ds(16 * j, 16)] = p0 * xs[j]
          orow[e, pl.ds(H, 16)] = jnp.where(iot == 0, p0, zeros16)
          pacc = jnp.where(iot == u, p0, pacc)
        pvals[pl.ds(16 * g, 16)] = pacc
      pltpu.sync_copy(orow, acc_sh.at[dstv], add=True)
      pltpu.sync_copy(pvals, p_out.at[pl.ds(off, CK)])

    fetch_a(0, srcvA, dstvA, xloA, xhiA, xrvA, semA)

    def chunk2_a(kk, carry):
      k = 2 * kk
      fetch_a(k + 1, srcvB, dstvB, xloB, xhiB, xrvB, semB)
      wait_a(srcvA, dstvA, xloA, xhiA, xrvA, semA)
      compute_a(k, dstvA, xloA, xhiA, xrvA)
      fetch_a(jnp.minimum(k + 2, last), srcvA, dstvA, xloA, xhiA, xrvA, semA)
      wait_a(srcvB, dstvB, xloB, xhiB, xrvB, semB)
      compute_a(k + 1, dstvB, xloB, xhiB, xrvB)
      return carry

    lax.fori_loop(0, n_chunks // 2, chunk2_a, 0)
    wait_a(srcvA, dstvA, xloA, xhiA, xrvA, semA)
    plsc.subcore_barrier()

    pltpu.sync_copy(acc_sh.at[pl.ds(tb, rows_per_tile)],
                    acc_lo_out.at[c, pl.ds(tb, rows_per_tile)])

    def zacc(k, carry):
      pltpu.sync_copy(zbuf, acc_sh.at[pl.ds(tb + 64 * k, 64)])
      return carry
    lax.fori_loop(0, n_zchunks, zacc, 0)
    plsc.subcore_barrier()

    lane_sel = [jnp.full((16,), u, jnp.int32) for u in range(16)]

    def fetch_b(i, srcv, dstv, xhiv, pv, sem):
      off = base + i * CK
      pltpu.sync_copy(src_hbm.at[pl.ds(off, CK)], srcv)
      pltpu.sync_copy(dst_hbm.at[pl.ds(off, CK)], dstv)
      pltpu.async_copy(xhi_hbm.at[srcv], xhiv, sem)
      pltpu.async_copy(p_out.at[pl.ds(off, CK)], pv, sem)

    def wait_b(srcv, xhiv, pv, sem):
      pltpu.make_async_copy(xhi_hbm.at[srcv], xhiv, sem).wait()
      pltpu.make_async_copy(p_out.at[pl.ds(0, CK)], pv, sem).wait()

    def compute_b(dstv, xhiv, pv):
      @plsc.parallel_loop(0, CK // 16)
      def group(g):
        pg = pv[pl.ds(16 * g, 16)]
        for u in range(16):
          e = 16 * g + u
          p0 = pg.at[lane_sel[u]].get(mode="promise_in_bounds")
          for j in range(4):
            orow[e, pl.ds(16 * j, 16)] = p0 * xhiv[e, pl.ds(16 * j, 16)]
      pltpu.sync_copy(orow, acc_sh.at[dstv], add=True)

    fetch_b(0, srcvA, dstvA, xhiA, pvA, semA)

    def chunk2_b(kk, carry):
      k = 2 * kk
      fetch_b(k + 1, srcvB, dstvB, xhiB, pvB, semB)
      wait_b(srcvA, xhiA, pvA, semA)
      compute_b(dstvA, xhiA, pvA)
      fetch_b(jnp.minimum(k + 2, last), srcvA, dstvA, xhiA, pvA, semA)
      wait_b(srcvB, xhiB, pvB, semB)
      compute_b(dstvB, xhiB, pvB)
      return carry

    lax.fori_loop(0, n_chunks // 2, chunk2_b, 0)
    wait_b(srcvA, xhiA, pvA, semA)
    plsc.subcore_barrier()

    pltpu.sync_copy(acc_sh.at[pl.ds(tb, rows_per_tile)],
                    acc_hi_out.at[c, pl.ds(tb, rows_per_tile)])

  return edge_pass


def _head_matmul(x, wa, wb):
  m = x.shape[0]
  bm = 1000
  grid = m // bm

  def body(x_ref, wa_ref, wb_ref, oa_ref, ob_ref):
    xv = x_ref[...]
    wa_v = wa_ref[...]
    wb_v = wb_ref[...]
    oa_ref[0] = jnp.dot(xv, wa_v[:, :H], preferred_element_type=jnp.float32)
    oa_ref[1] = jnp.dot(xv, wa_v[:, H:], preferred_element_type=jnp.float32)
    ob_ref[0] = jnp.dot(xv, wb_v[:, :H], preferred_element_type=jnp.float32)
    ob_ref[1] = jnp.dot(xv, wb_v[:, H:], preferred_element_type=jnp.float32)

  return pl.pallas_call(
      body,
      grid=(grid,),
      in_specs=[
          pl.BlockSpec((bm, F), lambda i: (i, 0)),
          pl.BlockSpec((F, F), lambda i: (0, 0)),
          pl.BlockSpec((F, F), lambda i: (0, 0)),
      ],
      out_specs=[
          pl.BlockSpec((2, bm, H), lambda i: (0, i, 0)),
          pl.BlockSpec((2, bm, H), lambda i: (0, i, 0)),
      ],
      out_shape=[
          jax.ShapeDtypeStruct((2, m, H), jnp.float32),
          jax.ShapeDtypeStruct((2, m, H), jnp.float32),
      ],
  )(x, wa, wb)


def _combine_matmul(acc, bias, wa, wb, m):
  bm = 1000
  grid = m // bm

  def body(acc_ref, b_ref, wa_ref, wb_ref, olo_ref, ohi_ref, ob_ref):
    a = jnp.concatenate([acc_ref[0, :, :H], acc_ref[1, :, :H]], axis=1)
    d0 = jnp.broadcast_to(acc_ref[0, :, H:H + 1], (bm, H))
    d1 = jnp.broadcast_to(acc_ref[1, :, H:H + 1], (bm, H))
    d = jnp.concatenate([d0, d1], axis=1)
    h = a / (d + 1e-16) + b_ref[...]
    h = jnp.maximum(h, 0.0)
    xl = jnp.dot(h, wa_ref[...], preferred_element_type=jnp.float32)
    olo_ref[...] = xl[:, :H]
    ohi_ref[...] = xl[:, H:]
    ob_ref[...] = jnp.dot(h, wb_ref[...], preferred_element_type=jnp.float32)

  return pl.pallas_call(
      body,
      grid=(grid,),
      in_specs=[
          pl.BlockSpec((2, bm, W), lambda i: (0, i, 0)),
          pl.BlockSpec((1, F), lambda i: (0, 0)),
          pl.BlockSpec((F, F), lambda i: (0, 0)),
          pl.BlockSpec((F, F), lambda i: (0, 0)),
      ],
      out_specs=[
          pl.BlockSpec((bm, H), lambda i: (i, 0)),
          pl.BlockSpec((bm, H), lambda i: (i, 0)),
          pl.BlockSpec((bm, F), lambda i: (i, 0)),
      ],
      out_shape=[
          jax.ShapeDtypeStruct((m, H), jnp.float32),
          jax.ShapeDtypeStruct((m, H), jnp.float32),
          jax.ShapeDtypeStruct((m, F), jnp.float32),
      ],
  )(acc, bias, wa, wb)


def _finale(acc_lo, acc_hi, bias, we1, be1, we2, be2, m):
  bm = 1000
  grid = m // bm

  def body(alo_ref, ahi_ref, b_ref, w1_ref, b1_ref, w2_ref, b2_ref, o_ref):
    a = jnp.concatenate([alo_ref[0, :, :H] + alo_ref[1, :, :H],
                         ahi_ref[0, :, :H] + ahi_ref[1, :, :H]], axis=1)
    d = alo_ref[0, :, H:H + 1] + alo_ref[1, :, H:H + 1]
    h = a / (jnp.broadcast_to(d, (bm, F)) + 1e-16) + b_ref[...]
    hmax = jnp.max(h, axis=1, keepdims=True)
    ex = jnp.exp(h - hmax)
    ls = h - hmax - jnp.log(jnp.sum(ex, axis=1, keepdims=True))
    r = jnp.dot(ls, w1_ref[...], preferred_element_type=jnp.float32) + b1_ref[...]
    r = jnp.maximum(r, 0.0)
    r = jnp.dot(r, w2_ref[...], preferred_element_type=jnp.float32) + b2_ref[...]
    r = jax.nn.sigmoid(r)
    o_ref[...] = ls + r

  return pl.pallas_call(
      body,
      grid=(grid,),
      in_specs=[
          pl.BlockSpec((2, bm, W), lambda i: (0, i, 0)),
          pl.BlockSpec((2, bm, W), lambda i: (0, i, 0)),
          pl.BlockSpec((1, F), lambda i: (0, 0)),
          pl.BlockSpec((F, F), lambda i: (0, 0)),
          pl.BlockSpec((1, F), lambda i: (0, 0)),
          pl.BlockSpec((F, F), lambda i: (0, 0)),
          pl.BlockSpec((1, F), lambda i: (0, 0)),
      ],
      out_specs=pl.BlockSpec((bm, F), lambda i: (i, 0)),
      out_shape=jax.ShapeDtypeStruct((m, F), jnp.float32),
  )(acc_lo, acc_hi, bias, we1, be1, we2, be2)


def kernel(x, edge_index, Wl1, Wr1, att1, b1, Wl2, Wr2, att2, b2,
           We1, be1, We2, be2):
  n = x.shape[0]
  e = edge_index.shape[1]
  n_edges = e + n
  e_pad = _cdiv(n_edges, NW * CK * 2) * (NW * CK * 2)
  n_rows = _cdiv(n + 1, NTILES * 64) * (NTILES * 64)

  loop = jnp.arange(n, dtype=jnp.int32)
  pad = e_pad - n_edges
  src = jnp.concatenate([edge_index[0], loop,
                         jnp.zeros((pad,), jnp.int32)])
  dst = jnp.concatenate([edge_index[1], loop,
                         jnp.full((pad,), n, jnp.int32)])

  edge1 = _make_edge_pass1(n_rows, e_pad, n)
  edge2 = _make_edge_pass2(n_rows, e_pad)

  xlh, xrh = _head_matmul(x, Wl1, Wr1)
  acc1 = edge1(xlh.reshape(2 * n, H), xrh.reshape(2 * n, H), att1, src, dst)
  xl2_lo, xl2_hi, xr2 = _combine_matmul(acc1[:, :n], b1.reshape(1, F),
                                        Wl2, Wr2, n)
  acc_lo, acc_hi, _ = edge2(xl2_lo, xl2_hi, xr2, att2.reshape(1, F), src, dst)
  return _finale(acc_lo[:, :n], acc_hi[:, :n], b2.reshape(1, F),
                 We1, be1.reshape(1, F), We2, be2.reshape(1, F), n)
